# baseline (speedup 1.0000x reference)
import functools

import jax
import jax.numpy as jnp
from jax import lax
from jax.experimental import pallas as pl
from jax.experimental.pallas import tpu as pltpu
from jax.experimental.pallas import tpu_sc as plsc

B = 16384
F = 26
VOCAB = 1000000
VPAD = 1000064

NC = 2
NS = 16
NW = NC * NS
CHUNK = B // NW
NIDX = CHUNK * F
L = 16


def _body(tf_hbm, xr_hbm, out_hbm, idx_v, vals_v, out_v, sem):
    wid = lax.axis_index("s") * NC + lax.axis_index("c")

    pltpu.sync_copy(xr_hbm.at[wid], idx_v)

    NH = 4
    HIDX = NIDX // NH
    HG = CHUNK // L // NH

    copies = [
        pltpu.async_copy(
            tf_hbm.at[idx_v.at[pl.ds(h * HIDX, HIDX)]],
            vals_v.at[pl.ds(h * HIDX, HIDX)],
            sem,
        )
        for h in range(NH)
    ]

    RPC = CHUNK // NH

    def make_g_body(h):
        def g_body(g, _):
            acc = jnp.zeros((L,), jnp.float32)
            for f in range(F):
                acc = acc + vals_v[pl.ds(h * HIDX + f * RPC + g * L, L)]
            out_v[pl.ds(h * RPC + g * L, L)] = 1.0 / (1.0 + jnp.exp(-acc))
            return _
        return g_body

    for h in range(NH):
        copies[h].wait()
        lax.fori_loop(0, HG, make_g_body(h), None)

    pltpu.sync_copy(out_v, out_hbm.at[pl.ds(wid * CHUNK, CHUNK)])


_sc_call = functools.partial(
    pl.kernel,
    out_type=jax.ShapeDtypeStruct((B,), jnp.float32),
    mesh=plsc.VectorSubcoreMesh(
        core_axis_name="c", subcore_axis_name="s",
        num_cores=NC, num_subcores=NS,
    ),
    compiler_params=pltpu.CompilerParams(needs_layout_passes=False),
    scratch_types=[
        pltpu.VMEM((NIDX,), jnp.int32),
        pltpu.VMEM((NIDX,), jnp.float32),
        pltpu.VMEM((CHUNK,), jnp.float32),
        pltpu.SemaphoreType.DMA,
    ],
)(_body)


@jax.jit
def kernel(x, table):
    xr = x.reshape(NW, 4, CHUNK // 4, F).transpose(0, 1, 3, 2).reshape(NW, NIDX)
    tt = jnp.pad(table, ((0, VPAD - VOCAB), (0, 0)))
    tt = tt.reshape(VPAD // 128, 128).reshape(VPAD)
    return _sc_call(tt, xr)


# speedup vs baseline: 1.0029x; 1.0029x over previous
import functools

import jax
import jax.numpy as jnp
from jax import lax
from jax.experimental import pallas as pl
from jax.experimental.pallas import tpu as pltpu
from jax.experimental.pallas import tpu_sc as plsc

B = 16384
F = 26
VOCAB = 1000000
VPAD = 1000064

NC = 2
NS = 16
NW = NC * NS
CHUNK = B // NW
NIDX = CHUNK * F
L = 16


def _body(tf_hbm, xr_hbm, out_hbm, idx_v, vals_v, out_v, sem):
    wid = lax.axis_index("s") * NC + lax.axis_index("c")

    pltpu.sync_copy(xr_hbm.at[wid], idx_v)

    pltpu.sync_copy(tf_hbm.at[idx_v], vals_v)

    def g_body(g, _):
        acc = jnp.zeros((L,), jnp.float32)
        for f in range(F):
            acc = acc + vals_v[pl.ds(f * CHUNK + g * L, L)]
        out_v[pl.ds(g * L, L)] = 1.0 / (1.0 + jnp.exp(-acc))
        return _

    lax.fori_loop(0, CHUNK // L, g_body, None)

    pltpu.sync_copy(out_v, out_hbm.at[pl.ds(wid * CHUNK, CHUNK)])


_sc_call = functools.partial(
    pl.kernel,
    out_type=jax.ShapeDtypeStruct((B,), jnp.float32),
    mesh=plsc.VectorSubcoreMesh(
        core_axis_name="c", subcore_axis_name="s",
        num_cores=NC, num_subcores=NS,
    ),
    compiler_params=pltpu.CompilerParams(needs_layout_passes=False),
    scratch_types=[
        pltpu.VMEM((NIDX,), jnp.int32),
        pltpu.VMEM((NIDX,), jnp.float32),
        pltpu.VMEM((CHUNK,), jnp.float32),
        pltpu.SemaphoreType.DMA,
    ],
)(_body)


@jax.jit
def kernel(x, table):
    xr = x.reshape(NW, CHUNK, F).transpose(0, 2, 1).reshape(NW, NIDX)
    tt = jnp.pad(table, ((0, VPAD - VOCAB), (0, 0))).reshape(VPAD)
    return _sc_call(tt, xr)


# speedup vs baseline: 1.0582x; 1.0551x over previous
import functools

import jax
import jax.numpy as jnp
from jax import lax
from jax.experimental import pallas as pl
from jax.experimental.pallas import tpu as pltpu
from jax.experimental.pallas import tpu_sc as plsc

B = 16384
F = 26
VOCAB = 1000000
VPAD = 1000064

NC = 2
NS = 16
NW = NC * NS
CHUNK = B // NW
NIDX = CHUNK * F
L = 16


def _body(tf_hbm, xt_hbm, out_hbm, idx_v, vals_v, out_v, sem):
    wid = lax.axis_index("s") * NC + lax.axis_index("c")

    stage = [
        pltpu.async_copy(
            xt_hbm.at[f, pl.ds(wid * CHUNK, CHUNK)],
            idx_v.at[pl.ds(f * CHUNK, CHUNK)],
            sem,
        )
        for f in range(F)
    ]
    for c in stage:
        c.wait()

    pltpu.sync_copy(tf_hbm.at[idx_v], vals_v)

    def g_body(g, _):
        acc = jnp.zeros((L,), jnp.float32)
        for f in range(F):
            acc = acc + vals_v[pl.ds(f * CHUNK + g * L, L)]
        out_v[pl.ds(g * L, L)] = 1.0 / (1.0 + jnp.exp(-acc))
        return _

    lax.fori_loop(0, CHUNK // L, g_body, None)

    pltpu.sync_copy(out_v, out_hbm.at[pl.ds(wid * CHUNK, CHUNK)])


_sc_call = functools.partial(
    pl.kernel,
    out_type=jax.ShapeDtypeStruct((B,), jnp.float32),
    mesh=plsc.VectorSubcoreMesh(
        core_axis_name="c", subcore_axis_name="s",
        num_cores=NC, num_subcores=NS,
    ),
    compiler_params=pltpu.CompilerParams(needs_layout_passes=False),
    scratch_types=[
        pltpu.VMEM((NIDX,), jnp.int32),
        pltpu.VMEM((NIDX,), jnp.float32),
        pltpu.VMEM((CHUNK,), jnp.float32),
        pltpu.SemaphoreType.DMA,
    ],
)(_body)


@jax.jit
def kernel(x, table):
    tt = jnp.pad(table, ((0, VPAD - VOCAB), (0, 0))).reshape(VPAD)
    return _sc_call(tt, x.T)


# speedup vs baseline: 1.0594x; 1.0011x over previous
import functools

import jax
import jax.numpy as jnp
from jax import lax
from jax.experimental import pallas as pl
from jax.experimental.pallas import tpu as pltpu
from jax.experimental.pallas import tpu_sc as plsc

B = 16384
F = 26
VOCAB = 1000000

NC = 2
NS = 16
NW = NC * NS
CHUNK = B // NW
NIDX = CHUNK * F
L = 16


def _body(tf_hbm, xt_hbm, out_hbm, idx_v, vals_v, out_v, sem):
    wid = lax.axis_index("s") * NC + lax.axis_index("c")

    stage = [
        pltpu.async_copy(
            xt_hbm.at[f, pl.ds(wid * CHUNK, CHUNK)],
            idx_v.at[pl.ds(f * CHUNK, CHUNK)],
            sem,
        )
        for f in range(F)
    ]
    for c in stage:
        c.wait()

    pltpu.sync_copy(tf_hbm.at[idx_v], vals_v)

    def g_body(g, _):
        acc = jnp.zeros((L,), jnp.float32)
        for f in range(F):
            acc = acc + vals_v[pl.ds(f * CHUNK + g * L, L)]
        out_v[pl.ds(g * L, L)] = 1.0 / (1.0 + jnp.exp(-acc))
        return _

    lax.fori_loop(0, CHUNK // L, g_body, None)

    pltpu.sync_copy(out_v, out_hbm.at[pl.ds(wid * CHUNK, CHUNK)])


_sc_call = functools.partial(
    pl.kernel,
    out_type=jax.ShapeDtypeStruct((B,), jnp.float32),
    mesh=plsc.VectorSubcoreMesh(
        core_axis_name="c", subcore_axis_name="s",
        num_cores=NC, num_subcores=NS,
    ),
    compiler_params=pltpu.CompilerParams(needs_layout_passes=False),
    scratch_types=[
        pltpu.VMEM((NIDX,), jnp.int32),
        pltpu.VMEM((NIDX,), jnp.float32),
        pltpu.VMEM((CHUNK,), jnp.float32),
        pltpu.SemaphoreType.DMA,
    ],
)(_body)


@jax.jit
def kernel(x, table):
    return _sc_call(table.reshape(VOCAB), x.T)
